# baseline (speedup 1.0000x reference)
import functools

import jax
import jax.numpy as jnp
from jax import lax
from jax.experimental import pallas as pl
from jax.experimental.pallas import tpu as pltpu
from jax.experimental.pallas import tpu_sc as plsc

BETA_C = 0.2
GAMMA_C = 0.95
N_N = 10000
N_E = 160000
D_F = 256

DC = 128
EC = 40
NB = 40
NBLK = N_N // NB
QB = 80
QBLK = N_N // QB
EPT = N_E // 16
ECH = EPT // EC
NBUF = 3

_mesh = plsc.VectorSubcoreMesh(core_axis_name="c", subcore_axis_name="s")

_scratch = dict(
    acc=pltpu.VMEM_SHARED((N_N, DC), jnp.float32),
    zsh=pltpu.VMEM_SHARED((N_N,), jnp.float32),
    zloc=pltpu.VMEM((N_N,), jnp.float32),
    zrows=pltpu.VMEM((16, D_F), jnp.float32),
    zmg=pltpu.VMEM((16,), jnp.float32),
    ztb=pltpu.VMEM((48,), jnp.float32),
)
for _b in range(NBUF):
    _scratch[f"eb0_{_b}"] = pltpu.VMEM((EC, DC), jnp.float32)
    _scratch[f"eb1_{_b}"] = pltpu.VMEM((EC, DC), jnp.float32)
    _scratch[f"src_{_b}"] = pltpu.VMEM((48,), jnp.int32)
    _scratch[f"dst_{_b}"] = pltpu.VMEM((EC,), jnp.int32)
    _scratch[f"isem_{_b}"] = pltpu.SemaphoreType.DMA
    _scratch[f"ssem_{_b}"] = pltpu.SemaphoreType.DMA


@functools.partial(
    pl.kernel,
    out_type=jax.ShapeDtypeStruct((N_N, D_F), jnp.float32),
    mesh=_mesh,
    compiler_params=pltpu.CompilerParams(needs_layout_passes=False),
    scratch_types=_scratch,
)
def _gnn_sc(z_hbm, e_hbm, src_hbm, dst_hbm, out_hbm, *, acc, zsh,
            zloc, zrows, zmg, ztb, **bufs):
    cid = lax.axis_index("c")
    sid = lax.axis_index("s")
    lane = lax.iota(jnp.int32, 16)
    zero16 = jnp.zeros((16,), jnp.float32)
    ebs0 = [bufs[f"eb0_{b}"] for b in range(NBUF)]
    ebs1 = [bufs[f"eb1_{b}"] for b in range(NBUF)]
    srcs = [bufs[f"src_{b}"] for b in range(NBUF)]
    dsts = [bufs[f"dst_{b}"] for b in range(NBUF)]
    isems = [bufs[f"isem_{b}"] for b in range(NBUF)]
    ssems = [bufs[f"ssem_{b}"] for b in range(NBUF)]
    col0 = DC * cid

    for b in range(NBUF):
        srcs[b][pl.ds(32, 16)] = jnp.zeros((16,), jnp.int32)

    n_z16 = ((N_N // 16) - sid + 15) // 16

    def zmax_blk(i, _):
        r0 = (sid + 16 * i) * 16
        pltpu.sync_copy(z_hbm.at[pl.ds(r0, 16)], zrows)

        def rowred(r, m):
            v = zrows[r, pl.ds(0, 16)]
            for j in range(1, D_F // 16):
                v = jnp.maximum(v, zrows[r, pl.ds(16 * j, 16)])
            return jnp.where(lane == r, jnp.max(v), m)

        m = lax.fori_loop(
            0, 16, rowred, jnp.full((16,), -jnp.inf, jnp.float32))
        zmg[pl.ds(0, 16)] = GAMMA_C * m
        pltpu.sync_copy(zmg, zsh.at[pl.ds(r0, 16)])
        return 0

    lax.fori_loop(0, n_z16, zmax_blk, 0)

    plsc.subcore_barrier()
    pltpu.sync_copy(zsh, zloc)

    n_my_blk = (NBLK - sid + 15) // 16
    n_my_qblk = (QBLK - sid + 15) // 16

    for p in range(2):
        is_q = p == 0

        def in_start(ch, b, is_q=is_q):
            e0 = sid * EPT + ch * EC
            pltpu.async_copy(dst_hbm.at[pl.ds(e0, EC)], dsts[b], isems[b])
            pltpu.async_copy(
                e_hbm.at[pl.ds(e0, EC), pl.ds(D_F + col0, DC)],
                ebs1[b], isems[b])
            if is_q:
                pltpu.async_copy(
                    src_hbm.at[pl.ds(e0, EC)], srcs[b].at[pl.ds(0, EC)],
                    isems[b])
                pltpu.async_copy(
                    e_hbm.at[pl.ds(e0, EC), pl.ds(col0, DC)],
                    ebs0[b], isems[b])

        def in_wait(ch, b, is_q=is_q):
            e0 = sid * EPT + ch * EC
            pltpu.make_async_copy(
                dst_hbm.at[pl.ds(e0, EC)], dsts[b], isems[b]).wait()
            pltpu.make_async_copy(
                e_hbm.at[pl.ds(e0, EC), pl.ds(D_F + col0, DC)],
                ebs1[b], isems[b]).wait()
            if is_q:
                pltpu.make_async_copy(
                    src_hbm.at[pl.ds(e0, EC)], srcs[b].at[pl.ds(0, EC)],
                    isems[b]).wait()
                pltpu.make_async_copy(
                    e_hbm.at[pl.ds(e0, EC), pl.ds(col0, DC)],
                    ebs0[b], isems[b]).wait()

        def sc_start(b, is_q=is_q):
            eb = ebs0[b] if is_q else ebs1[b]
            pltpu.async_copy(eb, acc.at[dsts[b]], ssems[b], add=True)

        def sc_wait(b, is_q=is_q):
            eb = ebs0[b] if is_q else ebs1[b]
            pltpu.make_async_copy(eb, acc.at[dsts[b]], ssems[b]).wait()

        plsc.subcore_barrier()

        def zfill(r, _):
            for j in range(DC // 16):
                ebs1[-1][r, pl.ds(16 * j, 16)] = zero16
            return 0

        lax.fori_loop(0, EC, zfill, 0)

        def zero_blk(i, _):
            r0 = (sid + 16 * i) * NB
            pltpu.sync_copy(ebs1[-1], acc.at[pl.ds(r0, NB)])
            return 0

        lax.fori_loop(0, n_my_blk, zero_blk, 0)
        plsc.subcore_barrier()

        def compute(b):
            for g in range(3):
                src16 = srcs[b][pl.ds(g * 16, 16)]
                ztb[pl.ds(g * 16, 16)] = plsc.load_gather(zloc, [src16])

            def erow(r, _, b=b):
                ziv = plsc.load_gather(
                    ztb, [jnp.full((16,), r, jnp.int32)])
                for j in range(DC // 16):
                    sl = pl.ds(16 * j, 16)
                    ebs0[b][r, sl] = ziv * ebs1[b][r, sl] + ebs0[b][r, sl]
                return 0

            lax.fori_loop(0, EC, erow, 0, unroll=2)

        in_start(0, 0)

        def ring(g3, _, is_q=is_q):
            for b in range(NBUF):
                ch = g3 * NBUF + b

                @pl.when(ch < ECH)
                def _(b=b, ch=ch):
                    nb = (b + 1) % NBUF

                    @pl.when(ch >= 2)
                    def _():
                        sc_wait(nb)

                    @pl.when(ch + 1 < ECH)
                    def _():
                        in_start(ch + 1, nb)

                    in_wait(ch, b)
                    if is_q:
                        compute(b)
                    sc_start(b)

            return 0

        lax.fori_loop(0, (ECH + NBUF - 1) // NBUF, ring, 0)
        sc_wait((ECH - 2) % NBUF)
        sc_wait((ECH - 1) % NBUF)

        plsc.subcore_barrier()

        if is_q:
            def qstage_blk(i, _):
                r0 = (sid + 16 * i) * QB
                pltpu.sync_copy(
                    acc.at[pl.ds(r0, QB)],
                    out_hbm.at[pl.ds(r0, QB), pl.ds(col0, DC)])
                return 0

            lax.fori_loop(0, n_my_qblk, qstage_blk, 0)
        else:
            qa, ab, zb, obuf = ebs0[0], ebs0[1], ebs0[2], ebs1[0]

            def drain_blk(i, _):
                r0 = (sid + 16 * i) * NB
                pltpu.sync_copy(
                    out_hbm.at[pl.ds(r0, NB), pl.ds(col0, DC)], qa)
                pltpu.sync_copy(acc.at[pl.ds(r0, NB)], ab)
                pltpu.sync_copy(
                    z_hbm.at[pl.ds(r0, NB), pl.ds(col0, DC)], zb)

                def comb_row(r, _):
                    for j in range(DC // 16):
                        sl = pl.ds(16 * j, 16)
                        obuf[r, sl] = (BETA_C * zb[r, sl]
                                       + (1.0 - BETA_C) * qa[r, sl]
                                       / (ab[r, sl] + 1e-6))
                    return 0

                lax.fori_loop(0, NB, comb_row, 0)
                pltpu.sync_copy(
                    obuf, out_hbm.at[pl.ds(r0, NB), pl.ds(col0, DC)])
                return 0

            lax.fori_loop(0, n_my_blk, drain_blk, 0)


def kernel(z, e, edge_index):
    e2 = e.reshape(N_E, 2 * D_F)
    ei = edge_index.astype(jnp.int32)
    return _gnn_sc(z, e2, ei[0], ei[1])


# speedup vs baseline: 1.0540x; 1.0540x over previous
import functools

import jax
import jax.numpy as jnp
from jax import lax
from jax.experimental import pallas as pl
from jax.experimental.pallas import tpu as pltpu
from jax.experimental.pallas import tpu_sc as plsc

BETA_C = 0.2
GAMMA_C = 0.95
N_N = 10000
N_E = 160000
D_F = 256

DC = 64
EC = 80
NB = 80
NBLK = N_N // NB
EPT = N_E // 16
ECH = EPT // EC
N_PASS = (D_F // DC) // 2
NBUF = 3

_mesh = plsc.VectorSubcoreMesh(core_axis_name="c", subcore_axis_name="s")

_scratch = dict(
    accq=pltpu.VMEM_SHARED((N_N, DC), jnp.float32),
    accac=pltpu.VMEM_SHARED((N_N, DC), jnp.float32),
    zsh=pltpu.VMEM_SHARED((N_N,), jnp.float32),
    zloc=pltpu.VMEM((N_N,), jnp.float32),
    zrows=pltpu.VMEM((16, D_F), jnp.float32),
    zmg=pltpu.VMEM((16,), jnp.float32),
    ztb=pltpu.VMEM((EC,), jnp.float32),
)
for _b in range(NBUF):
    _scratch[f"eb0_{_b}"] = pltpu.VMEM((EC, DC), jnp.float32)
    _scratch[f"eb1_{_b}"] = pltpu.VMEM((EC, DC), jnp.float32)
    _scratch[f"src_{_b}"] = pltpu.VMEM((EC,), jnp.int32)
    _scratch[f"dst_{_b}"] = pltpu.VMEM((EC,), jnp.int32)
    _scratch[f"isem_{_b}"] = pltpu.SemaphoreType.DMA
    _scratch[f"ssem_{_b}"] = pltpu.SemaphoreType.DMA


@functools.partial(
    pl.kernel,
    out_type=jax.ShapeDtypeStruct((N_N, D_F), jnp.float32),
    mesh=_mesh,
    compiler_params=pltpu.CompilerParams(
        use_tc_tiling_on_sc=False, needs_layout_passes=False),
    scratch_types=_scratch,
)
def _gnn_sc(z_hbm, e_hbm, src_hbm, dst_hbm, out_hbm, *, accq, accac, zsh,
            zloc, zrows, zmg, ztb, **bufs):
    cid = lax.axis_index("c")
    sid = lax.axis_index("s")
    lane = lax.iota(jnp.int32, 16)
    zero16 = jnp.zeros((16,), jnp.float32)
    ebs0 = [bufs[f"eb0_{b}"] for b in range(NBUF)]
    ebs1 = [bufs[f"eb1_{b}"] for b in range(NBUF)]
    srcs = [bufs[f"src_{b}"] for b in range(NBUF)]
    dsts = [bufs[f"dst_{b}"] for b in range(NBUF)]
    isems = [bufs[f"isem_{b}"] for b in range(NBUF)]
    ssems = [bufs[f"ssem_{b}"] for b in range(NBUF)]

    n_z16 = ((N_N // 16) - sid + 15) // 16

    def zmax_blk(i, _):
        r0 = (sid + 16 * i) * 16
        pltpu.sync_copy(z_hbm.at[pl.ds(r0, 16)], zrows)

        def rowred(r, m):
            v = zrows[r, pl.ds(0, 16)]
            for j in range(1, D_F // 16):
                v = jnp.maximum(v, zrows[r, pl.ds(16 * j, 16)])
            return jnp.where(lane == r, jnp.max(v), m)

        m = lax.fori_loop(
            0, 16, rowred, jnp.full((16,), -jnp.inf, jnp.float32))
        zmg[pl.ds(0, 16)] = GAMMA_C * m
        pltpu.sync_copy(zmg, zsh.at[pl.ds(r0, 16)])
        return 0

    lax.fori_loop(0, n_z16, zmax_blk, 0)

    plsc.subcore_barrier()
    pltpu.sync_copy(zsh, zloc)

    n_my_blk = (NBLK - sid + 15) // 16

    for p in range(N_PASS):
        k = 2 * p + cid
        col0 = DC * k

        def in_copies(ch, b):
            e0 = sid * EPT + ch * EC
            return (
                pltpu.make_async_copy(
                    src_hbm.at[pl.ds(e0, EC)], srcs[b], isems[b]),
                pltpu.make_async_copy(
                    dst_hbm.at[pl.ds(e0, EC)], dsts[b], isems[b]),
                pltpu.make_async_copy(
                    e_hbm.at[pl.ds(e0, EC), pl.ds(col0, DC)],
                    ebs0[b], isems[b]),
                pltpu.make_async_copy(
                    e_hbm.at[pl.ds(e0, EC), pl.ds(D_F + col0, DC)],
                    ebs1[b], isems[b]),
            )

        def sc_start(b):
            pltpu.async_copy(ebs0[b], accq.at[dsts[b]], ssems[b], add=True)
            pltpu.async_copy(ebs1[b], accac.at[dsts[b]], ssems[b], add=True)

        def sc_wait(b):
            pltpu.make_async_copy(ebs0[b], accq.at[dsts[b]], ssems[b]).wait()
            pltpu.make_async_copy(ebs1[b], accac.at[dsts[b]], ssems[b]).wait()

        plsc.subcore_barrier()

        def zfill(r, _):
            for j in range(DC // 16):
                ebs1[-1][r, pl.ds(16 * j, 16)] = zero16
            return 0

        lax.fori_loop(0, EC, zfill, 0)

        def zero_blk(i, _):
            r0 = (sid + 16 * i) * NB
            pltpu.sync_copy(ebs1[-1], accq.at[pl.ds(r0, NB)])
            pltpu.sync_copy(ebs1[-1], accac.at[pl.ds(r0, NB)])
            return 0

        lax.fori_loop(0, n_my_blk, zero_blk, 0)
        plsc.subcore_barrier()

        def compute(b):
            for g in range(EC // 16):
                src16 = srcs[b][pl.ds(g * 16, 16)]
                ztb[pl.ds(g * 16, 16)] = plsc.load_gather(zloc, [src16])

            def erow(r, _, b=b):
                ziv = plsc.load_gather(
                    ztb, [jnp.full((16,), r, jnp.int32)])
                for j in range(DC // 16):
                    sl = pl.ds(16 * j, 16)
                    ebs0[b][r, sl] = ziv * ebs1[b][r, sl] + ebs0[b][r, sl]
                return 0

            lax.fori_loop(0, EC, erow, 0, unroll=2)

        for c in in_copies(0, 0):
            c.start()

        def ring(g3, _):
            for b in range(NBUF):
                ch = g3 * NBUF + b

                @pl.when(ch < ECH)
                def _(b=b, ch=ch):
                    nb = (b + 1) % NBUF

                    @pl.when(ch >= 2)
                    def _():
                        sc_wait(nb)

                    @pl.when(ch + 1 < ECH)
                    def _():
                        for c in in_copies(ch + 1, nb):
                            c.start()

                    for c in in_copies(ch, b):
                        c.wait()
                    compute(b)
                    sc_start(b)

            return 0

        lax.fori_loop(0, (ECH + NBUF - 1) // NBUF, ring, 0)
        sc_wait((ECH - 2) % NBUF)
        sc_wait((ECH - 1) % NBUF)

        plsc.subcore_barrier()

        qa0, qa1, zb, obuf = ebs0[0], ebs0[1], ebs0[2], ebs1[0]

        def drain_blk(i, _):
            r0 = (sid + 16 * i) * NB
            pltpu.sync_copy(accq.at[pl.ds(r0, NB)], qa0)
            pltpu.sync_copy(accac.at[pl.ds(r0, NB)], qa1)
            pltpu.sync_copy(z_hbm.at[pl.ds(r0, NB), pl.ds(col0, DC)], zb)

            def comb_row(r, _):
                for j in range(DC // 16):
                    sl = pl.ds(16 * j, 16)
                    obuf[r, sl] = (BETA_C * zb[r, sl]
                                   + (1.0 - BETA_C) * qa0[r, sl]
                                   / (qa1[r, sl] + 1e-6))
                return 0

            lax.fori_loop(0, NB, comb_row, 0)
            pltpu.sync_copy(obuf, out_hbm.at[pl.ds(r0, NB), pl.ds(col0, DC)])
            return 0

        lax.fori_loop(0, n_my_blk, drain_blk, 0)


def kernel(z, e, edge_index):
    e2 = e.reshape(N_E, 2 * D_F)
    ei = edge_index.astype(jnp.int32)
    return _gnn_sc(z, e2, ei[0], ei[1])


# speedup vs baseline: 1.1849x; 1.1241x over previous
import functools

import jax
import jax.numpy as jnp
from jax import lax
from jax.experimental import pallas as pl
from jax.experimental.pallas import tpu as pltpu
from jax.experimental.pallas import tpu_sc as plsc

BETA_C = 0.2
GAMMA_C = 0.95
N_N = 10000
N_E = 160000
D_F = 256

DC = 64
EC = 80
NB = 80
NBLK = N_N // NB
EPT = N_E // 16
ECH = EPT // EC
N_PASS = (D_F // DC) // 2
NBUF = 3

_mesh = plsc.VectorSubcoreMesh(core_axis_name="c", subcore_axis_name="s")

_scratch = dict(
    acc=pltpu.VMEM_SHARED((N_N, 2 * DC), jnp.float32),
    zsh=pltpu.VMEM_SHARED((N_N,), jnp.float32),
    zloc=pltpu.VMEM((N_N,), jnp.float32),
    zrows=pltpu.VMEM((16, D_F), jnp.float32),
    zmg=pltpu.VMEM((16,), jnp.float32),
    ztb=pltpu.VMEM((EC,), jnp.float32),
)
for _b in range(NBUF):
    _scratch[f"eb_{_b}"] = pltpu.VMEM((EC, 2 * DC), jnp.float32)
    _scratch[f"src_{_b}"] = pltpu.VMEM((EC,), jnp.int32)
    _scratch[f"dst_{_b}"] = pltpu.VMEM((EC,), jnp.int32)
    _scratch[f"isem_{_b}"] = pltpu.SemaphoreType.DMA
    _scratch[f"ssem_{_b}"] = pltpu.SemaphoreType.DMA


@functools.partial(
    pl.kernel,
    out_type=jax.ShapeDtypeStruct((N_N, D_F), jnp.float32),
    mesh=_mesh,
    compiler_params=pltpu.CompilerParams(
        use_tc_tiling_on_sc=False, needs_layout_passes=False),
    scratch_types=_scratch,
)
def _gnn_sc(z_hbm, e_hbm, src_hbm, dst_hbm, out_hbm, *, acc, zsh,
            zloc, zrows, zmg, ztb, **bufs):
    cid = lax.axis_index("c")
    sid = lax.axis_index("s")
    lane = lax.iota(jnp.int32, 16)
    zero16 = jnp.zeros((16,), jnp.float32)
    ebs = [bufs[f"eb_{b}"] for b in range(NBUF)]
    srcs = [bufs[f"src_{b}"] for b in range(NBUF)]
    dsts = [bufs[f"dst_{b}"] for b in range(NBUF)]
    isems = [bufs[f"isem_{b}"] for b in range(NBUF)]
    ssems = [bufs[f"ssem_{b}"] for b in range(NBUF)]

    n_z16 = ((N_N // 16) - sid + 15) // 16

    def zmax_blk(i, _):
        r0 = (sid + 16 * i) * 16
        pltpu.sync_copy(z_hbm.at[pl.ds(r0, 16)], zrows)

        def rowred(r, m):
            v = zrows[r, pl.ds(0, 16)]
            for j in range(1, D_F // 16):
                v = jnp.maximum(v, zrows[r, pl.ds(16 * j, 16)])
            return jnp.where(lane == r, jnp.max(v), m)

        m = lax.fori_loop(
            0, 16, rowred, jnp.full((16,), -jnp.inf, jnp.float32))
        zmg[pl.ds(0, 16)] = GAMMA_C * m
        pltpu.sync_copy(zmg, zsh.at[pl.ds(r0, 16)])
        return 0

    lax.fori_loop(0, n_z16, zmax_blk, 0)

    plsc.subcore_barrier()
    pltpu.sync_copy(zsh, zloc)

    n_my_blk = (NBLK - sid + 15) // 16

    for p in range(N_PASS):
        k = 2 * p + cid
        col0 = DC * k

        def in_copies(ch, b):
            e0 = sid * EPT + ch * EC
            return (
                pltpu.make_async_copy(
                    src_hbm.at[pl.ds(e0, EC)], srcs[b], isems[b]),
                pltpu.make_async_copy(
                    dst_hbm.at[pl.ds(e0, EC)], dsts[b], isems[b]),
                pltpu.make_async_copy(
                    e_hbm.at[pl.ds(e0, EC), pl.ds(col0, DC)],
                    ebs[b].at[:, pl.ds(0, DC)], isems[b]),
                pltpu.make_async_copy(
                    e_hbm.at[pl.ds(e0, EC), pl.ds(D_F + col0, DC)],
                    ebs[b].at[:, pl.ds(DC, DC)], isems[b]),
            )

        def sc_start(b):
            pltpu.async_copy(ebs[b], acc.at[dsts[b]], ssems[b], add=True)

        def sc_wait(b):
            pltpu.make_async_copy(ebs[b], acc.at[dsts[b]], ssems[b]).wait()

        plsc.subcore_barrier()

        def zfill(r, _):
            for j in range(2 * DC // 16):
                ebs[-1][r, pl.ds(16 * j, 16)] = zero16
            return 0

        lax.fori_loop(0, EC, zfill, 0)

        def zero_blk(i, _):
            r0 = (sid + 16 * i) * NB
            pltpu.sync_copy(ebs[-1], acc.at[pl.ds(r0, NB)])
            return 0

        lax.fori_loop(0, n_my_blk, zero_blk, 0)
        plsc.subcore_barrier()

        def compute(b):
            for g in range(EC // 16):
                src16 = srcs[b][pl.ds(g * 16, 16)]
                ztb[pl.ds(g * 16, 16)] = plsc.load_gather(zloc, [src16])

            def erow(r, _, b=b):
                ziv = plsc.load_gather(
                    ztb, [jnp.full((16,), r, jnp.int32)])
                for j in range(DC // 16):
                    sl = pl.ds(16 * j, 16)
                    s1 = pl.ds(DC + 16 * j, 16)
                    ebs[b][r, sl] = ziv * ebs[b][r, s1] + ebs[b][r, sl]
                return 0

            lax.fori_loop(0, EC, erow, 0, unroll=2)

        for c in in_copies(0, 0):
            c.start()

        def ring(g3, _):
            for b in range(NBUF):
                ch = g3 * NBUF + b

                @pl.when(ch < ECH)
                def _(b=b, ch=ch):
                    nb = (b + 1) % NBUF

                    @pl.when(ch >= 2)
                    def _():
                        sc_wait(nb)

                    @pl.when(ch + 1 < ECH)
                    def _():
                        for c in in_copies(ch + 1, nb):
                            c.start()

                    for c in in_copies(ch, b):
                        c.wait()
                    compute(b)
                    sc_start(b)

            return 0

        lax.fori_loop(0, (ECH + NBUF - 1) // NBUF, ring, 0)
        sc_wait((ECH - 2) % NBUF)
        sc_wait((ECH - 1) % NBUF)

        plsc.subcore_barrier()

        qacb = ebs[0]
        zview = ebs[1].at[:, pl.ds(0, DC)]
        oview = ebs[1].at[:, pl.ds(DC, DC)]

        def drain_blk(i, _):
            r0 = (sid + 16 * i) * NB
            pltpu.sync_copy(acc.at[pl.ds(r0, NB)], qacb)
            pltpu.sync_copy(z_hbm.at[pl.ds(r0, NB), pl.ds(col0, DC)], zview)

            def comb_row(r, _):
                for j in range(DC // 16):
                    sl = pl.ds(16 * j, 16)
                    s1 = pl.ds(DC + 16 * j, 16)
                    ebs[1][r, s1] = (BETA_C * ebs[1][r, sl]
                                     + (1.0 - BETA_C) * qacb[r, sl]
                                     / (qacb[r, s1] + 1e-6))
                return 0

            lax.fori_loop(0, NB, comb_row, 0)
            pltpu.sync_copy(oview, out_hbm.at[pl.ds(r0, NB), pl.ds(col0, DC)])
            return 0

        lax.fori_loop(0, n_my_blk, drain_blk, 0)


def kernel(z, e, edge_index):
    e2 = e.reshape(N_E, 2 * D_F)
    ei = edge_index.astype(jnp.int32)
    return _gnn_sc(z, e2, ei[0], ei[1])


# speedup vs baseline: 1.1957x; 1.0092x over previous
import functools

import jax
import jax.numpy as jnp
from jax import lax
from jax.experimental import pallas as pl
from jax.experimental.pallas import tpu as pltpu
from jax.experimental.pallas import tpu_sc as plsc

BETA_C = 0.2
GAMMA_C = 0.95
N_N = 10000
N_E = 160000
D_F = 256

DC = 64
EC = 80
NB = 80
NBLK = N_N // NB
EPT = N_E // 16
ECH = EPT // EC
N_PASS = (D_F // DC) // 2
NBUF = 3

_mesh = plsc.VectorSubcoreMesh(core_axis_name="c", subcore_axis_name="s")

_scratch = dict(
    acc=pltpu.VMEM_SHARED((N_N, 2 * DC), jnp.float32),
    zsh=pltpu.VMEM_SHARED((N_N,), jnp.float32),
    zloc=pltpu.VMEM((N_N,), jnp.float32),
    zrows=pltpu.VMEM((16, D_F), jnp.float32),
    zmg=pltpu.VMEM((16,), jnp.float32),
    ztb=pltpu.VMEM((EC,), jnp.float32),
)
for _b in range(NBUF):
    _scratch[f"eb_{_b}"] = pltpu.VMEM((EC, 2 * DC), jnp.float32)
    _scratch[f"src_{_b}"] = pltpu.VMEM((EC,), jnp.int32)
    _scratch[f"dst_{_b}"] = pltpu.VMEM((EC,), jnp.int32)
    _scratch[f"isem_{_b}"] = pltpu.SemaphoreType.DMA
    _scratch[f"ssem_{_b}"] = pltpu.SemaphoreType.DMA


@functools.partial(
    pl.kernel,
    out_type=jax.ShapeDtypeStruct((N_N, D_F), jnp.float32),
    mesh=_mesh,
    compiler_params=pltpu.CompilerParams(
        use_tc_tiling_on_sc=False, needs_layout_passes=False),
    scratch_types=_scratch,
)
def _gnn_sc(z_hbm, e_hbm, src_hbm, dst_hbm, out_hbm, *, acc, zsh,
            zloc, zrows, zmg, ztb, **bufs):
    cid = lax.axis_index("c")
    sid = lax.axis_index("s")
    lane = lax.iota(jnp.int32, 16)
    zero16 = jnp.zeros((16,), jnp.float32)
    ebs = [bufs[f"eb_{b}"] for b in range(NBUF)]
    srcs = [bufs[f"src_{b}"] for b in range(NBUF)]
    dsts = [bufs[f"dst_{b}"] for b in range(NBUF)]
    isems = [bufs[f"isem_{b}"] for b in range(NBUF)]
    ssems = [bufs[f"ssem_{b}"] for b in range(NBUF)]

    n_z16 = ((N_N // 16) - sid + 15) // 16

    def zmax_blk(i, _):
        r0 = (sid + 16 * i) * 16
        pltpu.sync_copy(z_hbm.at[pl.ds(r0, 16)], zrows)

        def rowred(r, m):
            v = zrows[r, pl.ds(0, 16)]
            for j in range(1, D_F // 16):
                v = jnp.maximum(v, zrows[r, pl.ds(16 * j, 16)])
            return jnp.where(lane == r, jnp.max(v), m)

        m = lax.fori_loop(
            0, 16, rowred, jnp.full((16,), -jnp.inf, jnp.float32))
        zmg[pl.ds(0, 16)] = GAMMA_C * m
        pltpu.sync_copy(zmg, zsh.at[pl.ds(r0, 16)])
        return 0

    lax.fori_loop(0, n_z16, zmax_blk, 0)

    plsc.subcore_barrier()
    pltpu.sync_copy(zsh, zloc)

    n_my_blk = (NBLK - sid + 15) // 16

    for p in range(N_PASS):
        k = 2 * p + cid
        col0 = DC * k

        def in_copies(ch, b):
            e0 = sid * EPT + ch * EC
            return (
                pltpu.make_async_copy(
                    src_hbm.at[pl.ds(e0, EC)], srcs[b], isems[b]),
                pltpu.make_async_copy(
                    dst_hbm.at[pl.ds(e0, EC)], dsts[b], isems[b]),
                pltpu.make_async_copy(
                    e_hbm.at[pl.ds(e0, EC), pl.ds(col0, DC)],
                    ebs[b].at[:, pl.ds(0, DC)], isems[b]),
                pltpu.make_async_copy(
                    e_hbm.at[pl.ds(e0, EC), pl.ds(D_F + col0, DC)],
                    ebs[b].at[:, pl.ds(DC, DC)], isems[b]),
            )

        def sc_start(b):
            pltpu.async_copy(ebs[b], acc.at[dsts[b]], ssems[b], add=True)

        def sc_wait(b):
            pltpu.make_async_copy(ebs[b], acc.at[dsts[b]], ssems[b]).wait()

        plsc.subcore_barrier()

        def zfill(r, _):
            for j in range(2 * DC // 16):
                ebs[-1][r, pl.ds(16 * j, 16)] = zero16
            return 0

        lax.fori_loop(0, EC, zfill, 0)

        def zero_blk(i, _):
            r0 = (sid + 16 * i) * NB
            pltpu.sync_copy(ebs[-1], acc.at[pl.ds(r0, NB)])
            return 0

        lax.fori_loop(0, n_my_blk, zero_blk, 0)
        plsc.subcore_barrier()

        def compute(b):
            for g in range(EC // 16):
                src16 = srcs[b][pl.ds(g * 16, 16)]
                ztb[pl.ds(g * 16, 16)] = plsc.load_gather(zloc, [src16])

            def erow(r, _, b=b):
                ziv = plsc.load_gather(
                    ztb, [jnp.full((16,), r, jnp.int32)])
                for j in range(DC // 16):
                    sl = pl.ds(16 * j, 16)
                    s1 = pl.ds(DC + 16 * j, 16)
                    ebs[b][r, sl] = ziv * ebs[b][r, s1] + ebs[b][r, sl]
                return 0

            lax.fori_loop(0, EC, erow, 0, unroll=4)

        for c in in_copies(0, 0):
            c.start()

        def ring(g3, _):
            for b in range(NBUF):
                ch = g3 * NBUF + b

                @pl.when(ch < ECH)
                def _(b=b, ch=ch):
                    nb = (b + 1) % NBUF

                    @pl.when(ch >= 2)
                    def _():
                        sc_wait(nb)

                    @pl.when(ch + 1 < ECH)
                    def _():
                        for c in in_copies(ch + 1, nb):
                            c.start()

                    for c in in_copies(ch, b):
                        c.wait()
                    compute(b)
                    sc_start(b)

            return 0

        lax.fori_loop(0, (ECH + NBUF - 1) // NBUF, ring, 0)
        sc_wait((ECH - 2) % NBUF)
        sc_wait((ECH - 1) % NBUF)

        plsc.subcore_barrier()

        qacb = ebs[0]
        zview = ebs[1].at[:, pl.ds(0, DC)]
        oview = ebs[1].at[:, pl.ds(DC, DC)]

        def drain_blk(i, _):
            r0 = (sid + 16 * i) * NB
            pltpu.sync_copy(acc.at[pl.ds(r0, NB)], qacb)
            pltpu.sync_copy(z_hbm.at[pl.ds(r0, NB), pl.ds(col0, DC)], zview)

            def comb_row(r, _):
                for j in range(DC // 16):
                    sl = pl.ds(16 * j, 16)
                    s1 = pl.ds(DC + 16 * j, 16)
                    ebs[1][r, s1] = (BETA_C * ebs[1][r, sl]
                                     + (1.0 - BETA_C) * qacb[r, sl]
                                     / (qacb[r, s1] + 1e-6))
                return 0

            lax.fori_loop(0, NB, comb_row, 0)
            pltpu.sync_copy(oview, out_hbm.at[pl.ds(r0, NB), pl.ds(col0, DC)])
            return 0

        lax.fori_loop(0, n_my_blk, drain_blk, 0)


def kernel(z, e, edge_index):
    e2 = e.reshape(N_E, 2 * D_F)
    ei = edge_index.astype(jnp.int32)
    return _gnn_sc(z, e2, ei[0], ei[1])
